# Initial kernel scaffold; baseline (speedup 1.0000x reference)
import jax
import jax.numpy as jnp
from jax.experimental import pallas as pl


def kernel(pre_prob, diags, procs, syms, C_diag, C_proc, C_sym, c1_high_limit, c1_low_limit, c1_minus_weight, c1_plus_weight):
    raise NotImplementedError("write your pallas kernel here")


# speedup vs baseline: 3.9603x; 3.9603x over previous
import functools

import jax
import jax.numpy as jnp
from jax import lax
from jax.experimental import pallas as pl
from jax.experimental.pallas import tpu as pltpu
from jax.experimental.pallas import tpu_sc as plsc

N_MED = 1000
G = 128
L = 16
N_IDX = 16
N_FULL = 7
TAIL_BASE = N_FULL * G
TAIL_W = N_MED - TAIL_BASE
N_D, N_P, N_S = 16, 8, 12
N_ROWS = N_D + N_P + N_S


def _body(diag_hbm, proc_hbm, sym_hbm, d_idx_hbm, p_idx_hbm, s_idx_hbm,
          pre_hbm, par_hbm, out_hbm,
          d_idx_v, p_idx_v, s_idx_v, dbuf, pbuf, sbuf, pre_v, par_v, out_v,
          tail_buf, sem):
    wid = lax.axis_index("s") * 2 + lax.axis_index("c")

    @pl.when(wid <= N_FULL)
    def _active():
        pltpu.sync_copy(d_idx_hbm, d_idx_v)
        pltpu.sync_copy(p_idx_hbm, p_idx_v)
        pltpu.sync_copy(s_idx_hbm, s_idx_v)
        pltpu.sync_copy(par_hbm, par_v)

        hi0 = par_v[0, :]
        hi1 = par_v[1, :]
        hi2 = par_v[2, :]
        lo0 = par_v[3, :]
        lo1 = par_v[4, :]
        lo2 = par_v[5, :]
        neg_w = par_v[6, :]
        pos_w = par_v[7, :]
        zero = jnp.zeros((L,), jnp.float32)
        one = jnp.ones((L,), jnp.float32)

        def colmax(buf, c):
            m = buf[0, pl.ds(c, L)]
            for r in range(1, N_IDX):
                m = jnp.maximum(m, buf[r, pl.ds(c, L)])
            return m

        def compute(chunks):
            for c in chunks:
                md = colmax(dbuf, c)
                mp = colmax(pbuf, c)
                ms = colmax(sbuf, c)
                low = (jnp.where(md < lo0, one, zero)
                       * jnp.where(mp < lo1, one, zero)
                       * jnp.where(ms < lo2, one, zero))
                anyhigh = jnp.maximum(
                    jnp.where(md > hi0, one, zero),
                    jnp.maximum(jnp.where(mp > hi1, one, zero),
                                jnp.where(ms > hi2, one, zero)))
                high = (one - low) * anyhigh
                adj = neg_w * low + pos_w * high
                out_v[pl.ds(c, L)] = pre_v[pl.ds(c, L)] + adj

        @pl.when(wid < N_FULL)
        def _full():
            base = pl.multiple_of(wid * G, G)
            pltpu.sync_copy(pre_hbm.at[pl.ds(base, G)], pre_v)
            cp_d = pltpu.async_copy(
                diag_hbm.at[d_idx_v, pl.ds(base, G)], dbuf, sem)
            cp_p = pltpu.async_copy(
                proc_hbm.at[p_idx_v, pl.ds(base, G)], pbuf, sem)
            cp_s = pltpu.async_copy(
                sym_hbm.at[s_idx_v, pl.ds(base, G)], sbuf, sem)
            cp_d.wait()
            cp_p.wait()
            cp_s.wait()
            compute(tuple(range(0, G, L)))
            pltpu.sync_copy(out_v, out_hbm.at[pl.ds(base, G)])

        @pl.when(wid == N_FULL)
        def _tail():
            pltpu.sync_copy(pre_hbm.at[pl.ds(TAIL_BASE, TAIL_W)],
                            pre_v.at[pl.ds(0, TAIL_W)])
            lane = lax.iota(jnp.int32, L)
            izero = jnp.zeros((L,), jnp.int32)

            def extract(vec, r):
                fvec = plsc.bitcast(jnp.where(lane == r, vec, izero),
                                    jnp.float32)
                return lax.bitcast_convert_type(jnp.max(fvec), jnp.int32)

            def fetch(tbl_hbm, idx_vec, n, buf_off):
                cps = []
                for r in range(n):
                    s = extract(vec=idx_vec, r=r)
                    t = pl.multiple_of((s >> 3) << 3, 8)
                    cps.append(pltpu.async_copy(
                        tbl_hbm.at[pl.ds(t, 8), pl.ds(TAIL_BASE, TAIL_W)],
                        tail_buf.at[buf_off + r], sem))
                return cps

            d_vec = d_idx_v[...]
            p_vec = p_idx_v[...]
            s_vec = s_idx_v[...]
            cps = (fetch(diag_hbm, d_vec, N_D, 0)
                   + fetch(proc_hbm, p_vec, N_P, N_D)
                   + fetch(sym_hbm, s_vec, N_S, N_D + N_P))
            for cp in cps:
                cp.wait()

            offs_d = d_vec & 7
            offs_p = p_vec & 7
            offs_s = s_vec & 7
            chunks = tuple(range(0, TAIL_W - L + 1, L)) + (TAIL_W - L,)
            for c in chunks:
                col_idx = lane + c

                def rowmax(offs, n, buf_off):
                    m = None
                    for r in range(n):
                        o = jnp.full((L,), extract(offs, r), jnp.int32)
                        row = plsc.load_gather(
                            tail_buf,
                            [jnp.full((L,), buf_off + r, jnp.int32), o,
                             col_idx])
                        m = row if m is None else jnp.maximum(m, row)
                    return m

                md = rowmax(offs_d, N_D, 0)
                mp = rowmax(offs_p, N_P, N_D)
                ms = rowmax(offs_s, N_S, N_D + N_P)
                low = (jnp.where(md < lo0, one, zero)
                       * jnp.where(mp < lo1, one, zero)
                       * jnp.where(ms < lo2, one, zero))
                anyhigh = jnp.maximum(
                    jnp.where(md > hi0, one, zero),
                    jnp.maximum(jnp.where(mp > hi1, one, zero),
                                jnp.where(ms > hi2, one, zero)))
                high = (one - low) * anyhigh
                adj = neg_w * low + pos_w * high
                out_v[pl.ds(c, L)] = pre_v[pl.ds(c, L)] + adj
            pltpu.sync_copy(out_v.at[pl.ds(0, TAIL_W)],
                            out_hbm.at[pl.ds(TAIL_BASE, TAIL_W)])

    return None


_sc_review = functools.partial(
    pl.kernel,
    out_type=jax.ShapeDtypeStruct((N_MED,), jnp.float32),
    mesh=plsc.VectorSubcoreMesh(core_axis_name="c", subcore_axis_name="s"),
    compiler_params=pltpu.CompilerParams(needs_layout_passes=False),
    scratch_types=[
        pltpu.VMEM((N_IDX,), jnp.int32),
        pltpu.VMEM((N_IDX,), jnp.int32),
        pltpu.VMEM((N_IDX,), jnp.int32),
        pltpu.VMEM((N_IDX, G), jnp.float32),
        pltpu.VMEM((N_IDX, G), jnp.float32),
        pltpu.VMEM((N_IDX, G), jnp.float32),
        pltpu.VMEM((G,), jnp.float32),
        pltpu.VMEM((8, L), jnp.float32),
        pltpu.VMEM((G,), jnp.float32),
        pltpu.VMEM((N_ROWS, 8, TAIL_W), jnp.float32),
        pltpu.SemaphoreType.DMA,
    ],
)(_body)


@jax.jit
def kernel(pre_prob, diags, procs, syms, C_diag, C_proc, C_sym,
           c1_high_limit, c1_low_limit, c1_minus_weight, c1_plus_weight):
    d_idx = diags.astype(jnp.int32)
    p_idx = procs.astype(jnp.int32)[jnp.arange(N_IDX) % procs.shape[0]]
    s_idx = syms.astype(jnp.int32)[jnp.arange(N_IDX) % syms.shape[0]]
    par = jnp.stack([
        c1_high_limit[0], c1_high_limit[1], c1_high_limit[2],
        c1_low_limit[0], c1_low_limit[1], c1_low_limit[2],
        -c1_minus_weight, c1_plus_weight,
    ]).astype(jnp.float32)
    par = jnp.broadcast_to(par[:, None], (8, L))
    pre = pre_prob.reshape(N_MED)
    out = _sc_review(C_diag, C_proc, C_sym, d_idx, p_idx, s_idx, pre, par)
    return out.reshape(1, N_MED)


# speedup vs baseline: 4.3147x; 1.0895x over previous
import functools

import jax
import jax.numpy as jnp
from jax import lax
from jax.experimental import pallas as pl
from jax.experimental.pallas import tpu as pltpu
from jax.experimental.pallas import tpu_sc as plsc

N_MED = 1000
L = 16
MT = N_MED // 8
W = 32
NW = 32
LAST_OFF = N_MED - W
N_D, N_P, N_S = 16, 8, 12
NPAR = 16
NBUF = 4

N_DIAG, N_PROC, N_SYM = 20000, 10000, 5000
NT_D, EW_D = N_DIAG // 128, N_DIAG % 128
NT_P, EW_P = N_PROC // 128, N_PROC % 128
NT_S, EW_S = N_SYM // 128, N_SYM % 128


def _body(diag_hbm, proc_hbm, sym_hbm, d_idx_hbm, p_idx_hbm, s_idx_hbm,
          par_hbm, out_hbm,
          d_idx_v, p_idx_v, s_idx_v, b0, b1, b2, b3, ed, ep, es, par_v,
          out_v, sem):
    wid = lax.axis_index("s") * 2 + lax.axis_index("c")
    ring = (b0, b1, b2, b3)

    pltpu.sync_copy(d_idx_hbm, d_idx_v)
    pltpu.sync_copy(p_idx_hbm, p_idx_v)
    pltpu.sync_copy(s_idx_hbm, s_idx_v)
    pltpu.sync_copy(par_hbm, par_v)

    out_off = jnp.minimum(wid * W, LAST_OFF)
    out_off = pl.multiple_of(out_off, 8)
    mt0 = out_off >> 3

    lane = lax.iota(jnp.int32, L)
    izero = jnp.zeros((L,), jnp.int32)
    zero = jnp.zeros((L,), jnp.float32)
    one = jnp.ones((L,), jnp.float32)

    d_vec = plsc.load_gather(d_idx_v, [lane])
    p_vec = plsc.load_gather(
        p_idx_v, [lax.rem(lane, jnp.full((L,), N_P, jnp.int32))])
    s_vec = plsc.load_gather(
        s_idx_v, [lax.rem(lane, jnp.full((L,), N_S, jnp.int32))])

    def extract(vec, r):
        fvec = plsc.bitcast(jnp.where(lane == r, vec, izero), jnp.float32)
        return lax.bitcast_convert_type(jnp.max(fvec), jnp.int32)

    cp_ed = pltpu.async_copy(
        diag_hbm.at[pl.ds(mt0, 4), :, pl.ds(NT_D * 128, EW_D)], ed, sem)
    cp_ep = pltpu.async_copy(
        proc_hbm.at[pl.ds(mt0, 4), :, pl.ds(NT_P * 128, EW_P)], ep, sem)
    cp_es = pltpu.async_copy(
        sym_hbm.at[pl.ds(mt0, 4), :, pl.ds(NT_S * 128, EW_S)], es, sem)

    strips = ([(diag_hbm, d_vec, r, NT_D, ed, EW_D) for r in range(N_D)]
              + [(proc_hbm, p_vec, r, NT_P, ep, EW_P) for r in range(N_P)]
              + [(sym_hbm, s_vec, r, NT_S, es, EW_S) for r in range(N_S)])
    n_strips = len(strips)

    def fire(k):
        tbl, vec, r, nt, _, _ = strips[k]
        ct = extract(vec, r) >> 7
        ct = jnp.where(ct >= nt, nt - 1, ct)
        col0 = pl.multiple_of(ct << 7, 128)
        return pltpu.async_copy(
            tbl.at[pl.ds(mt0, 4), :, pl.ds(col0, 128)], ring[k % NBUF],
            sem)

    acc = [[zero, zero], [zero, zero], [zero, zero]]
    tbl_of = [0] * N_D + [1] * N_P + [2] * N_S

    def process(k):
        tbl, vec, r, nt, ebuf, ew = strips[k]
        idxb = jnp.full((L,), extract(vec, r), jnp.int32)
        c_n = idxb & 127
        c_e = idxb & (ew - 1)
        is_edge = idxb >= (nt << 7)
        buf = ring[k % NBUF]
        t = tbl_of[k]
        for ci, c0 in enumerate((0, L)):
            m_loc = lane + c0
            mt_loc = m_loc >> 3
            rr = m_loc & 7
            v_n = plsc.load_gather(buf, [mt_loc, rr, c_n])
            v_e = plsc.load_gather(ebuf, [mt_loc, rr, c_e])
            v = jnp.where(is_edge, v_e, v_n)
            acc[t][ci] = jnp.maximum(acc[t][ci], v)

    cps = [fire(k) for k in range(NBUF)]
    cp_ed.wait()
    cp_ep.wait()
    cp_es.wait()
    for k in range(n_strips):
        cps[k].wait()
        if k + NBUF < n_strips:
            cps.append(fire(k + NBUF))
        process(k)

    def bcast(j):
        return plsc.load_gather(par_v, [jnp.full((L,), j, jnp.int32)])

    hi = (bcast(8), bcast(9), bcast(10))
    lo = (bcast(11), bcast(12), bcast(13))
    neg_w, pos_w = bcast(14), bcast(15)

    for ci, c0 in enumerate((0, L)):
        md, mp, ms = acc[0][ci], acc[1][ci], acc[2][ci]
        low = (jnp.where(md < lo[0], one, zero)
               * jnp.where(mp < lo[1], one, zero)
               * jnp.where(ms < lo[2], one, zero))
        anyhigh = jnp.maximum(
            jnp.where(md > hi[0], one, zero),
            jnp.maximum(jnp.where(mp > hi[1], one, zero),
                        jnp.where(ms > hi[2], one, zero)))
        high = (one - low) * anyhigh
        adj = neg_w * low + pos_w * high
        pre = par_v[pl.ds(NPAR + out_off + c0, L)]
        out_v[pl.ds(c0, L)] = pre + adj

    pltpu.sync_copy(out_v, out_hbm.at[pl.ds(out_off, W)])
    return None


_sc_review = functools.partial(
    pl.kernel,
    out_type=jax.ShapeDtypeStruct((N_MED,), jnp.float32),
    mesh=plsc.VectorSubcoreMesh(core_axis_name="c", subcore_axis_name="s"),
    compiler_params=pltpu.CompilerParams(needs_layout_passes=False),
    scratch_types=[
        pltpu.VMEM((N_D,), jnp.int32),
        pltpu.VMEM((N_P,), jnp.int32),
        pltpu.VMEM((N_S,), jnp.int32),
        pltpu.VMEM((4, 8, 128), jnp.float32),
        pltpu.VMEM((4, 8, 128), jnp.float32),
        pltpu.VMEM((4, 8, 128), jnp.float32),
        pltpu.VMEM((4, 8, 128), jnp.float32),
        pltpu.VMEM((4, 8, EW_D), jnp.float32),
        pltpu.VMEM((4, 8, EW_P), jnp.float32),
        pltpu.VMEM((4, 8, EW_S), jnp.float32),
        pltpu.VMEM((NPAR + N_MED,), jnp.float32),
        pltpu.VMEM((W,), jnp.float32),
        pltpu.SemaphoreType.DMA,
    ],
)(_body)


@jax.jit
def kernel(pre_prob, diags, procs, syms, C_diag, C_proc, C_sym,
           c1_high_limit, c1_low_limit, c1_minus_weight, c1_plus_weight):
    par = jnp.concatenate([
        jnp.zeros((8,), jnp.float32),
        c1_high_limit.astype(jnp.float32),
        c1_low_limit.astype(jnp.float32),
        jnp.stack([-c1_minus_weight, c1_plus_weight]).astype(jnp.float32),
        pre_prob.reshape(N_MED),
    ])
    out = _sc_review(C_diag.T.reshape(MT, 8, N_DIAG),
                     C_proc.T.reshape(MT, 8, N_PROC),
                     C_sym.T.reshape(MT, 8, N_SYM),
                     diags.astype(jnp.int32), procs.astype(jnp.int32),
                     syms.astype(jnp.int32), par)
    return out.reshape(1, N_MED)


# speedup vs baseline: 16.3388x; 3.7868x over previous
import functools

import jax
import jax.numpy as jnp
from jax import lax
from jax.experimental import pallas as pl
from jax.experimental.pallas import tpu as pltpu
from jax.experimental.pallas import tpu_sc as plsc

N_MED = 1000
L = 16
MT = N_MED // 8
W = 32
NW = 32
LAST_OFF = N_MED - W
N_D, N_P, N_S = 16, 8, 12
NPAR = 16
NBUF = 4

N_DIAG, N_PROC, N_SYM = 20000, 10000, 5000
NT_D, EW_D = N_DIAG // 128, N_DIAG % 128
NT_P, EW_P = N_PROC // 128, N_PROC % 128
N_FULL_MED = N_MED // 128
EW_M = N_MED % 128


def _body(diag_hbm, proc_hbm, sym_hbm, d_idx_hbm, p_idx_hbm, s_idx_hbm,
          par_hbm, out_hbm,
          d_idx_v, p_idx_v, s_idx_v, b0, b1, b2, b3, ed, ep, sbufa, sbufb,
          sym_acc, par_v, out_v, sem, sem_e, sem_s):
    wid = lax.axis_index("s") * 2 + lax.axis_index("c")
    ring = (b0, b1, b2, b3)

    pltpu.sync_copy(d_idx_hbm, d_idx_v)
    pltpu.sync_copy(p_idx_hbm, p_idx_v)
    pltpu.sync_copy(s_idx_hbm, s_idx_v)
    pltpu.sync_copy(par_hbm, par_v)

    out_off = jnp.minimum(wid * W, LAST_OFF)
    out_off = pl.multiple_of(out_off, 8)
    mt0 = out_off >> 3

    lane = lax.iota(jnp.int32, L)
    izero = jnp.zeros((L,), jnp.int32)
    zero = jnp.zeros((L,), jnp.float32)
    one = jnp.ones((L,), jnp.float32)

    d_vec = plsc.load_gather(d_idx_v, [lane])
    p_vec = plsc.load_gather(
        p_idx_v, [lax.rem(lane, jnp.full((L,), N_P, jnp.int32))])
    s_vec = plsc.load_gather(
        s_idx_v, [lax.rem(lane, jnp.full((L,), N_S, jnp.int32))])

    def extract(vec, r):
        fvec = plsc.bitcast(jnp.where(lane == r, vec, izero), jnp.float32)
        return lax.bitcast_convert_type(jnp.max(fvec), jnp.int32)

    cp_ed = pltpu.async_copy(
        diag_hbm.at[pl.ds(mt0, 4), :, pl.ds(NT_D * 128, EW_D)], ed, sem_e)
    cp_ep = pltpu.async_copy(
        proc_hbm.at[pl.ds(mt0, 4), :, pl.ds(NT_P * 128, EW_P)], ep, sem_e)

    strips = ([(diag_hbm, d_vec, r, NT_D, ed, EW_D) for r in range(N_D)]
              + [(proc_hbm, p_vec, r, NT_P, ep, EW_P) for r in range(N_P)])
    n_strips = len(strips)

    def fire(k):
        tbl, vec, r, nt, _, _ = strips[k]
        ct = extract(vec, r) >> 7
        ct = jnp.where(ct >= nt, nt - 1, ct)
        col0 = pl.multiple_of(ct << 7, 128)
        return pltpu.async_copy(
            tbl.at[pl.ds(mt0, 4), :, pl.ds(col0, 128)], ring[k % NBUF],
            sem)

    acc = [[zero, zero], [zero, zero]]
    tbl_of = [0] * N_D + [1] * N_P

    def process(k):
        tbl, vec, r, nt, ebuf, ew = strips[k]
        idxb = jnp.full((L,), extract(vec, r), jnp.int32)
        c_n = idxb & 127
        c_e = idxb & (ew - 1)
        is_edge = idxb >= (nt << 7)
        buf = ring[k % NBUF]
        t = tbl_of[k]
        for ci, c0 in enumerate((0, L)):
            m_loc = lane + c0
            mt_loc = m_loc >> 3
            rr = m_loc & 7
            v_n = plsc.load_gather(buf, [mt_loc, rr, c_n])
            v_e = plsc.load_gather(ebuf, [mt_loc, rr, c_e])
            v = jnp.where(is_edge, v_e, v_n)
            acc[t][ci] = jnp.maximum(acc[t][ci], v)

    s_tile = [extract(s_vec, r) for r in range(N_S)]
    s_off = [jnp.full((L,), extract(s_vec & 7, r), jnp.int32)
             for r in range(N_S)]

    def sym_do(buf, colb, colb_vec):
        cps_s = [pltpu.async_copy(sym_hbm.at[s_tile[r] >> 3, :, colb],
                                  buf.at[r + 1], sem_s)
                 for r in range(N_S)]
        for cp in cps_s:
            cp.wait()
        for ci, c0 in enumerate((0, L)):
            col_in = lane + c0 + colb_vec
            m = None
            for r in range(N_S):
                v = plsc.load_gather(
                    buf, [jnp.full((L,), r + 1, jnp.int32), s_off[r],
                          col_in])
                m = v if m is None else jnp.maximum(m, v)
            sym_acc[ci, :] = m

    in_tail = out_off >= N_FULL_MED * 128

    @pl.when(jnp.logical_not(in_tail))
    def _sym_main():
        colb = pl.multiple_of((out_off >> 7) << 7, 128)
        sym_do(sbufa, pl.ds(colb, 128),
               jnp.full((L,), out_off - colb, jnp.int32))

    @pl.when(in_tail)
    def _sym_tail():
        sym_do(sbufb, pl.ds(N_FULL_MED * 128, N_MED - N_FULL_MED * 128),
               jnp.full((L,), out_off - N_FULL_MED * 128, jnp.int32))

    cps = [fire(k) for k in range(NBUF)]
    cp_ed.wait()
    cp_ep.wait()
    for k in range(n_strips):
        cps[k].wait()
        if k + NBUF < n_strips:
            cps.append(fire(k + NBUF))
        process(k)

    def bcast(j):
        return plsc.load_gather(par_v, [jnp.full((L,), j, jnp.int32)])

    hi = (bcast(8), bcast(9), bcast(10))
    lo = (bcast(11), bcast(12), bcast(13))
    neg_w, pos_w = bcast(14), bcast(15)

    for ci, c0 in enumerate((0, L)):
        md, mp, ms = acc[0][ci], acc[1][ci], sym_acc[ci, :]
        low = (jnp.where(md < lo[0], one, zero)
               * jnp.where(mp < lo[1], one, zero)
               * jnp.where(ms < lo[2], one, zero))
        anyhigh = jnp.maximum(
            jnp.where(md > hi[0], one, zero),
            jnp.maximum(jnp.where(mp > hi[1], one, zero),
                        jnp.where(ms > hi[2], one, zero)))
        high = (one - low) * anyhigh
        adj = neg_w * low + pos_w * high
        pre = par_v[pl.ds(NPAR + out_off + c0, L)]
        out_v[pl.ds(c0, L)] = pre + adj

    pltpu.sync_copy(out_v, out_hbm.at[pl.ds(out_off, W)])
    return None


_sc_review = functools.partial(
    pl.kernel,
    out_type=jax.ShapeDtypeStruct((N_MED,), jnp.float32),
    mesh=plsc.VectorSubcoreMesh(core_axis_name="c", subcore_axis_name="s"),
    compiler_params=pltpu.CompilerParams(needs_layout_passes=False),
    scratch_types=[
        pltpu.VMEM((N_D,), jnp.int32),
        pltpu.VMEM((N_P,), jnp.int32),
        pltpu.VMEM((N_S,), jnp.int32),
        pltpu.VMEM((4, 8, 128), jnp.float32),
        pltpu.VMEM((4, 8, 128), jnp.float32),
        pltpu.VMEM((4, 8, 128), jnp.float32),
        pltpu.VMEM((4, 8, 128), jnp.float32),
        pltpu.VMEM((4, 8, EW_D), jnp.float32),
        pltpu.VMEM((4, 8, EW_P), jnp.float32),
        pltpu.VMEM((N_S + 1, 8, 128), jnp.float32),
        pltpu.VMEM((N_S + 1, 8, EW_M), jnp.float32),
        pltpu.VMEM((2, L), jnp.float32),
        pltpu.VMEM((NPAR + N_MED,), jnp.float32),
        pltpu.VMEM((W,), jnp.float32),
        pltpu.SemaphoreType.DMA,
        pltpu.SemaphoreType.DMA,
        pltpu.SemaphoreType.DMA,
    ],
)(_body)


@jax.jit
def kernel(pre_prob, diags, procs, syms, C_diag, C_proc, C_sym,
           c1_high_limit, c1_low_limit, c1_minus_weight, c1_plus_weight):
    par = jnp.concatenate([
        jnp.zeros((8,), jnp.float32),
        c1_high_limit.astype(jnp.float32),
        c1_low_limit.astype(jnp.float32),
        jnp.stack([-c1_minus_weight, c1_plus_weight]).astype(jnp.float32),
        pre_prob.reshape(N_MED),
    ])
    out = _sc_review(C_diag.T.reshape(MT, 8, N_DIAG),
                     C_proc.T.reshape(MT, 8, N_PROC),
                     C_sym.reshape(N_SYM // 8, 8, N_MED),
                     diags.astype(jnp.int32), procs.astype(jnp.int32),
                     syms.astype(jnp.int32), par)
    return out.reshape(1, N_MED)
